# Initial kernel scaffold; baseline (speedup 1.0000x reference)
import jax
import jax.numpy as jnp
from jax.experimental import pallas as pl


def kernel(coordinates, plane_xy, plane_xz, plane_yz, proj_w, proj_b):
    raise NotImplementedError("write your pallas kernel here")


# speedup vs baseline: 23.4380x; 23.4380x over previous
import functools

import jax
import jax.numpy as jnp
from jax import lax
from jax.experimental import pallas as pl
from jax.experimental.pallas import tpu as pltpu
from jax.experimental.pallas import tpu_sc as plsc

RES = 512
RANK = 8
OUT = 8
PLANE_ROWS = RES * RES
TBL_ROWS = 3 * PLANE_ROWS
D = 4 * RANK

RB = 4096
NBLK = PLANE_ROWS // RB

NC = 2
NS = 16
NW = NC * NS
L = 16
B = 512
IDX_ROWS = 3 * B // 128


def _prep_body(pln_ref, pln2_ref, m_ref, b_ref, o_ref):
    i = pl.program_id(1)
    blk = pln_ref[0]
    nblk = pln2_ref[0]
    m = m_ref[0]
    grow = i * RB + lax.broadcasted_iota(jnp.int32, (RB, 1), 0)
    is_x_edge = (grow % RES) == (RES - 1)
    is_y_edge = ((grow // RES) % RES) == (RES - 1)
    bx = jnp.where(is_x_edge, blk, jnp.concatenate([blk[1:], blk[-1:]], axis=0))
    ay = jnp.concatenate([blk[RES:], nblk[:RES]], axis=0)
    ay = jnp.where(is_y_edge, blk, ay)
    axy = jnp.where(is_x_edge, ay, jnp.concatenate([ay[1:], ay[-1:]], axis=0))
    f32 = jnp.float32
    pp = lax.dot(blk, m, preferred_element_type=f32)
    ppx = lax.dot(bx, m, preferred_element_type=f32)
    ppy = lax.dot(ay, m, preferred_element_type=f32)
    ppxy = lax.dot(axy, m, preferred_element_type=f32)
    o_ref[...] = jnp.concatenate([pp, ppx, ppy, ppxy], axis=1) + b_ref[...]


def _tc_prep(planes3, m3, bvec4):
    return pl.pallas_call(
        _prep_body,
        grid=(3, NBLK),
        in_specs=[
            pl.BlockSpec((1, RB, RANK), lambda p, i: (p, i, 0)),
            pl.BlockSpec((1, RB, RANK), lambda p, i: (p, jnp.minimum(i + 1, NBLK - 1), 0)),
            pl.BlockSpec((1, RANK, RANK), lambda p, i: (p, 0, 0)),
            pl.BlockSpec((1, D), lambda p, i: (0, 0)),
        ],
        out_specs=pl.BlockSpec((RB, D), lambda p, i: (p * NBLK + i, 0)),
        out_shape=jax.ShapeDtypeStruct((TBL_ROWS, D), jnp.float32),
    )(planes3, planes3, m3, bvec4)


def _sc_interp(coords_flat, tbl):
    n = coords_flat.shape[0] // 3
    nt = (n + B - 1) // B
    jmax = (nt + NW - 1) // NW
    mesh = plsc.VectorSubcoreMesh(
        core_axis_name="c", subcore_axis_name="s", num_cores=NC, num_subcores=NS)

    @functools.partial(
        pl.kernel,
        out_type=jax.ShapeDtypeStruct((n * OUT,), jnp.float32),
        mesh=mesh,
        scratch_types=[
            pltpu.VMEM((3 * B,), jnp.float32),
            pltpu.VMEM((3 * B,), jnp.int32),
            pltpu.VMEM((3 * B,), jnp.float32),
            pltpu.VMEM((3 * B,), jnp.float32),
            pltpu.VMEM((3 * B, D), jnp.float32),
            pltpu.VMEM((B * OUT,), jnp.float32),
            pltpu.SemaphoreType.DMA,
        ],
        compiler_params=pltpu.CompilerParams(
            needs_layout_passes=False, use_tc_tiling_on_sc=False),
    )
    def sc_k(coords_hbm, tbl_hbm, out_hbm, cbuf, ibuf, wxbuf, wybuf, gbuf,
             obuf, gsem):
        wid = lax.axis_index("s") * NC + lax.axis_index("c")

        def batch_body(j, carry):
            t = wid + NW * j

            @pl.when(t < nt)
            def _():
                base = jnp.minimum(t * B, n - B)
                pltpu.sync_copy(coords_hbm.at[pl.ds(base * 3, B * 3)], cbuf)

                def idx_body(g, carry2):
                    i0 = g * L
                    rows = i0 + lax.iota(jnp.int32, L)
                    rows3 = rows * 3
                    for p, (cu, cv) in enumerate(((0, 1), (0, 2), (1, 2))):
                        u = plsc.load_gather(cbuf, [rows3 + cu])
                        v = plsc.load_gather(cbuf, [rows3 + cv])
                        uc = jnp.minimum(jnp.maximum(u, -1.0), 1.0)
                        iu = (uc + 1.0) * 256.0 - 0.5
                        iu = jnp.minimum(jnp.maximum(iu, 0.0), 511.0)
                        u0 = iu.astype(jnp.int32)
                        wx = iu - u0.astype(jnp.float32)
                        vc = jnp.minimum(jnp.maximum(v, -1.0), 1.0)
                        iv = (vc + 1.0) * 256.0 - 0.5
                        iv = jnp.minimum(jnp.maximum(iv, 0.0), 511.0)
                        v0 = iv.astype(jnp.int32)
                        wy = iv - v0.astype(jnp.float32)
                        idx = p * PLANE_ROWS + v0 * RES + u0
                        pos = p * B + i0
                        ibuf[pl.ds(pos, L)] = idx
                        wxbuf[pl.ds(pos, L)] = wx
                        wybuf[pl.ds(pos, L)] = wy
                    return carry2

                lax.fori_loop(0, B // L, idx_body, None)

                handles = []
                for r in range(IDX_ROWS):
                    handles.append(pltpu.async_copy(
                        tbl_hbm.at[ibuf.at[pl.ds(r * 128, 128)]],
                        gbuf.at[pl.ds(r * 128, 128)],
                        gsem))
                for h in handles:
                    h.wait()

                def int_body(g, carry2):
                    i0 = g * L
                    rows = i0 + lax.iota(jnp.int32, L)
                    ws = []
                    gbs = []
                    for p in range(3):
                        pos = p * B + i0
                        wx = wxbuf[pl.ds(pos, L)]
                        wy = wybuf[pl.ds(pos, L)]
                        w11 = wx * wy
                        w01 = wx - w11
                        w10 = wy - w11
                        w00 = (1.0 - wx) - w10
                        ws.append((w00, w01, w10, w11))
                        gbs.append(rows + p * B)
                    rows8 = rows * OUT
                    lanes = [jnp.full((L,), c, jnp.int32) for c in range(D)]
                    for c in range(OUT):
                        acc = None
                        for p in range(3):
                            w00, w01, w10, w11 = ws[p]
                            gb = gbs[p]
                            g00 = plsc.load_gather(gbuf, [gb, lanes[c]])
                            g01 = plsc.load_gather(gbuf, [gb, lanes[c + 8]])
                            g10 = plsc.load_gather(gbuf, [gb, lanes[c + 16]])
                            g11 = plsc.load_gather(gbuf, [gb, lanes[c + 24]])
                            term = w00 * g00 + w01 * g01 + w10 * g10 + w11 * g11
                            acc = term if acc is None else acc + term
                        acc = jnp.minimum(jnp.maximum(acc, -10.0), 10.0)
                        plsc.store_scatter(obuf, [rows8 + c], acc)
                    return carry2

                lax.fori_loop(0, B // L, int_body, None)
                pltpu.sync_copy(obuf, out_hbm.at[pl.ds(base * OUT, B * OUT)])

            return carry

        lax.fori_loop(0, jmax, batch_body, None)

    return sc_k(coords_flat, tbl)


def kernel(coordinates, plane_xy, plane_xz, plane_yz, proj_w, proj_b):
    planes3 = jnp.stack([
        plane_xy.reshape(PLANE_ROWS, RANK),
        plane_xz.reshape(PLANE_ROWS, RANK),
        plane_yz.reshape(PLANE_ROWS, RANK),
    ])
    m3 = jnp.transpose(proj_w.reshape(OUT, 3, RANK), (1, 2, 0))
    bvec4 = (jnp.tile(proj_b, 4) / 3.0)[None, :]
    tbl = _tc_prep(planes3, m3, bvec4)
    n = coordinates.shape[0]
    out_flat = _sc_interp(coordinates.reshape(n * 3), tbl)
    return out_flat.reshape(n, OUT)
